# baseline (speedup 1.0000x reference)
import functools

import jax
import jax.numpy as jnp
from jax import lax
from jax.experimental import pallas as pl
from jax.experimental.pallas import tpu as pltpu
from jax.experimental.pallas import tpu_sc as plsc

T = 2048
D = 768
E = 8
F = 3072
K = 2
A = T * K
BLK = 256
NB = A // BLK + E
NPAD = NB * BLK
PCH = 512

NW = 32
GCH = 96
CCH = 64


def _fiota(shape, dim):
    return lax.broadcasted_iota(jnp.int32, shape, dim).astype(jnp.float32)


def _router_body(x_ref, wg_ref, bg_ref, pos_ref, gj_ref, s_ref, x16_ref):
    xf = x_ref[...]
    x16_ref[...] = xf.astype(jnp.bfloat16)
    logits = lax.dot_general(
        wg_ref[...], xf, (((0,), (1,)), ((), ())),
        preferred_element_type=jnp.float32) + bg_ref[...]
    m = jnp.max(logits, axis=0, keepdims=True)
    ex = jnp.exp(logits - m)
    gates = ex / jnp.sum(ex, axis=0, keepdims=True)

    erow = _fiota( (E, T), 0)
    g1 = jnp.max(gates, axis=0, keepdims=True)
    i1 = jnp.min(jnp.where(gates == g1, erow, jnp.float32(E)),
                 axis=0, keepdims=True)
    gates2 = jnp.where(erow == i1, -jnp.inf, gates)
    g2 = jnp.max(gates2, axis=0, keepdims=True)
    i2 = jnp.min(jnp.where(gates2 == g2, erow, jnp.float32(E)),
                 axis=0, keepdims=True)

    ea = jnp.concatenate([i1, i2], axis=1)
    gj = jnp.concatenate([g1, g2], axis=1)

    erowA = _fiota( (E, A), 0)
    oh = (ea == erowA).astype(jnp.float32)
    cs = oh
    sh = 1
    while sh < A:
        cs = cs + jnp.concatenate(
            [jnp.zeros((E, sh), jnp.float32), cs[:, :-sh]], axis=1)
        sh *= 2
    counts = cs[:, A - 1:A]
    nblk = jnp.floor((counts + (BLK - 1)) * (1.0 / BLK))
    padded = nblk * BLK
    ltri = (_fiota( (E, E), 0)
            > _fiota( (E, E), 1)).astype(jnp.float32)
    off = jnp.dot(ltri, padded, preferred_element_type=jnp.float32)
    ends_blk = (off + padded) * (1.0 / BLK)

    posf = jnp.sum(oh * (off + cs - 1.0), axis=0, keepdims=True)
    posi = posf.astype(jnp.int32)
    pos_ref[0:1, :] = posi[:, :T]
    pos_ref[1:2, :] = posi[:, T:]
    gj_ref[0:1, :] = g1
    gj_ref[1:2, :] = g2

    bcol = _fiota( (1, NB), 1)
    be = jnp.sum((bcol >= ends_blk).astype(jnp.float32), axis=0, keepdims=True)
    erow_c = _fiota( (E, 1), 0)
    emax = jnp.max(jnp.where(counts > 0.0, erow_c, 0.0), axis=0,
                   keepdims=True)
    be = jnp.minimum(be, emax)
    total = jnp.max(ends_blk, axis=0, keepdims=True)
    s_ref[...] = jnp.concatenate([be, total], axis=1).astype(jnp.int32)


def _router_call(xf, Wg, bg):
    return pl.pallas_call(
        _router_body,
        out_shape=(
            jax.ShapeDtypeStruct((K, T), jnp.int32),
            jax.ShapeDtypeStruct((K, T), jnp.float32),
            jax.ShapeDtypeStruct((1, NB + 1), jnp.int32),
            jax.ShapeDtypeStruct((T, D), jnp.bfloat16),
        ),
    )(xf, Wg, bg.reshape(E, 1))


_SQRT_HALF = 0.7071067811865476


def _gemm_body(s_ref, x16_ref, pos_ref, w1_ref, b1_ref, w2_ref, b2_ref,
               o_ref):
    i = pl.program_id(0)

    @pl.when(i < s_ref[NB])
    def _active():
        slot = lax.broadcasted_iota(jnp.int32, (BLK, T), 0) + i * BLK
        p1 = lax.broadcast_in_dim(pos_ref[0:1, :], (BLK, T), (0, 1))
        p2 = lax.broadcast_in_dim(pos_ref[1:2, :], (BLK, T), (0, 1))
        p = ((p1 == slot) | (p2 == slot)).astype(jnp.bfloat16)
        xb = jnp.dot(p, x16_ref[...],
                     preferred_element_type=jnp.float32).astype(jnp.bfloat16)
        w1 = w1_ref[0].astype(jnp.bfloat16)
        h = jnp.dot(xb, w1, preferred_element_type=jnp.float32) + b1_ref[0]
        h = 0.5 * h * (1.0 + lax.erf(h * _SQRT_HALF))
        w2 = w2_ref[0].astype(jnp.bfloat16)
        o_ref[...] = jnp.dot(h.astype(jnp.bfloat16), w2,
                             preferred_element_type=jnp.float32) + b2_ref[0]


def _gemm_call(s, x16, pos2d, W1, b1, W2, b2):
    grid_spec = pltpu.PrefetchScalarGridSpec(
        num_scalar_prefetch=1,
        grid=(NB,),
        in_specs=[
            pl.BlockSpec((T, D), lambda i, s: (0, 0)),
            pl.BlockSpec((K, T), lambda i, s: (0, 0)),
            pl.BlockSpec((1, D, F), lambda i, s: (s[i], 0, 0)),
            pl.BlockSpec((1, 1, F), lambda i, s: (s[i], 0, 0)),
            pl.BlockSpec((1, F, D), lambda i, s: (s[i], 0, 0)),
            pl.BlockSpec((1, 1, D), lambda i, s: (s[i], 0, 0)),
        ],
        out_specs=pl.BlockSpec((BLK, D), lambda i, s: (i, 0)),
    )
    return pl.pallas_call(
        _gemm_body,
        grid_spec=grid_spec,
        out_shape=jax.ShapeDtypeStruct((NPAD, D), jnp.float32),
    )(s, x16, pos2d, W1, b1.reshape(E, 1, F), W2, b2.reshape(E, 1, D))


@functools.lru_cache(maxsize=None)
def _sc_combine():
    mesh = plsc.VectorSubcoreMesh(core_axis_name="c", subcore_axis_name="s")

    nch = 8
    ch = CCH // nch

    @functools.partial(
        pl.kernel,
        mesh=mesh,
        out_type=jax.ShapeDtypeStruct((T, D), jnp.float32),
        scratch_types=[
            pltpu.VMEM((CCH,), jnp.int32),
            pltpu.VMEM((CCH,), jnp.int32),
            pltpu.VMEM((CCH + 16,), jnp.float32),
            pltpu.VMEM((CCH + 16,), jnp.float32),
            pltpu.VMEM((CCH, D), jnp.float32),
            pltpu.VMEM((CCH, D), jnp.float32),
        ] + [pltpu.SemaphoreType.DMA] * (2 * nch),
    )
    def combine(h_hbm, pos_hbm, gj_hbm, out_hbm,
                i1_v, i2_v, g1_v, g2_v, a_v, b_v, *sems):
        wid = lax.axis_index("s") * 2 + lax.axis_index("c")
        base = wid * CCH
        pltpu.sync_copy(pos_hbm.at[0, pl.ds(base, CCH)], i1_v)
        pltpu.sync_copy(pos_hbm.at[1, pl.ds(base, CCH)], i2_v)
        copies = []
        for c in range(nch):
            sl = pl.ds(c * ch, ch)
            copies.append(pltpu.async_copy(
                h_hbm.at[i1_v.at[sl]], a_v.at[sl], sems[c]))
            copies.append(pltpu.async_copy(
                h_hbm.at[i2_v.at[sl]], b_v.at[sl], sems[nch + c]))
        pltpu.sync_copy(gj_hbm.at[0, pl.ds(base, CCH)], g1_v.at[pl.ds(0, CCH)])
        pltpu.sync_copy(gj_hbm.at[1, pl.ds(base, CCH)], g2_v.at[pl.ds(0, CCH)])
        for cp in copies:
            cp.wait()

        def row_add(i, carry):
            ga = g1_v[pl.ds(i, 16)][0]
            gb = g2_v[pl.ds(i, 16)][0]
            for k in range(D // 16):
                sl = pl.ds(k * 16, 16)
                a_v[i, sl] = a_v[i, sl] * ga + b_v[i, sl] * gb
            return carry

        lax.fori_loop(0, CCH, row_add, 0)
        pltpu.sync_copy(a_v, out_hbm.at[pl.ds(base, CCH)])

    return combine


def kernel(x, Wg, bg, W1, b1, W2, b2):
    b, t, d = x.shape
    xf = x.reshape(T, D)
    pos, gj, s, x16 = _router_call(xf, Wg, bg)
    H = _gemm_call(s.reshape(NB + 1), x16, pos, W1, b1, W2, b2)
    out = _sc_combine()(H, pos, gj)
    return out.reshape(b, t, d)


# speedup vs baseline: 1.0020x; 1.0020x over previous
import functools

import jax
import jax.numpy as jnp
from jax import lax
from jax.experimental import pallas as pl
from jax.experimental.pallas import tpu as pltpu
from jax.experimental.pallas import tpu_sc as plsc

T = 2048
D = 768
E = 8
F = 3072
K = 2
A = T * K
BLK = 256
NB = A // BLK + E
NPAD = NB * BLK
PCH = 512

NW = 32
GCH = 96
CCH = 64


def _fiota(shape, dim):
    return lax.broadcasted_iota(jnp.int32, shape, dim).astype(jnp.float32)


def _router_body(x_ref, wg_ref, bg_ref, pos_ref, gj_ref, s_ref, x16_ref):
    xf = x_ref[...]
    x16_ref[...] = xf.astype(jnp.bfloat16)
    logits = lax.dot_general(
        wg_ref[...], xf, (((0,), (1,)), ((), ())),
        preferred_element_type=jnp.float32) + bg_ref[...]
    m = jnp.max(logits, axis=0, keepdims=True)
    ex = jnp.exp(logits - m)
    gates = ex / jnp.sum(ex, axis=0, keepdims=True)

    erow = _fiota( (E, T), 0)
    g1 = jnp.max(gates, axis=0, keepdims=True)
    i1 = jnp.min(jnp.where(gates == g1, erow, jnp.float32(E)),
                 axis=0, keepdims=True)
    gates2 = jnp.where(erow == i1, -jnp.inf, gates)
    g2 = jnp.max(gates2, axis=0, keepdims=True)
    i2 = jnp.min(jnp.where(gates2 == g2, erow, jnp.float32(E)),
                 axis=0, keepdims=True)

    ea = jnp.concatenate([i1, i2], axis=1)
    gj = jnp.concatenate([g1, g2], axis=1)

    erowA = _fiota( (E, A), 0)
    oh = (ea == erowA).astype(jnp.float32)
    cs = oh
    sh = 1
    while sh < A:
        cs = cs + jnp.concatenate(
            [jnp.zeros((E, sh), jnp.float32), cs[:, :-sh]], axis=1)
        sh *= 2
    counts = cs[:, A - 1:A]
    nblk = jnp.floor((counts + (BLK - 1)) * (1.0 / BLK))
    padded = nblk * BLK
    ltri = (_fiota( (E, E), 0)
            > _fiota( (E, E), 1)).astype(jnp.float32)
    off = jnp.dot(ltri, padded, preferred_element_type=jnp.float32)
    ends_blk = (off + padded) * (1.0 / BLK)

    posf = jnp.sum(oh * (off + cs - 1.0), axis=0, keepdims=True)
    posi = posf.astype(jnp.int32)
    pos_ref[0:1, :] = posi[:, :T]
    pos_ref[1:2, :] = posi[:, T:]
    gj_ref[0:1, :] = g1
    gj_ref[1:2, :] = g2

    bcol = _fiota( (1, NB), 1)
    be = jnp.sum((bcol >= ends_blk).astype(jnp.float32), axis=0, keepdims=True)
    erow_c = _fiota( (E, 1), 0)
    emax = jnp.max(jnp.where(counts > 0.0, erow_c, 0.0), axis=0,
                   keepdims=True)
    be = jnp.minimum(be, emax)
    total = jnp.max(ends_blk, axis=0, keepdims=True)
    s_ref[...] = jnp.concatenate([be, total], axis=1).astype(jnp.int32)


def _router_call(xf, Wg, bg):
    return pl.pallas_call(
        _router_body,
        out_shape=(
            jax.ShapeDtypeStruct((K, T), jnp.int32),
            jax.ShapeDtypeStruct((K, T), jnp.float32),
            jax.ShapeDtypeStruct((1, NB + 1), jnp.int32),
            jax.ShapeDtypeStruct((T, D), jnp.bfloat16),
        ),
    )(xf, Wg, bg.reshape(E, 1))


_SQRT_HALF = 0.7071067811865476


def _gemm_body(s_ref, x16_ref, pos_ref, w1_ref, b1_ref, w2_ref, b2_ref,
               o_ref):
    i = pl.program_id(0)

    @pl.when(i < s_ref[NB])
    def _active():
        slot = lax.broadcasted_iota(jnp.int32, (BLK, T), 0) + i * BLK
        p1 = lax.broadcast_in_dim(pos_ref[0:1, :], (BLK, T), (0, 1))
        p2 = lax.broadcast_in_dim(pos_ref[1:2, :], (BLK, T), (0, 1))
        p = ((p1 == slot) | (p2 == slot)).astype(jnp.bfloat16)
        xb = jnp.dot(p, x16_ref[...],
                     preferred_element_type=jnp.float32).astype(jnp.bfloat16)
        w1 = w1_ref[0].astype(jnp.bfloat16)
        h = jnp.dot(xb, w1, preferred_element_type=jnp.float32) + b1_ref[0]
        h = 0.5 * h * (1.0 + lax.erf(h * _SQRT_HALF))
        w2 = w2_ref[0].astype(jnp.bfloat16)
        o_ref[...] = jnp.dot(h.astype(jnp.bfloat16), w2,
                             preferred_element_type=jnp.float32) + b2_ref[0]


def _gemm_call(s, x16, pos2d, W1, b1, W2, b2):
    grid_spec = pltpu.PrefetchScalarGridSpec(
        num_scalar_prefetch=1,
        grid=(NB,),
        in_specs=[
            pl.BlockSpec((T, D), lambda i, s: (0, 0)),
            pl.BlockSpec((K, T), lambda i, s: (0, 0)),
            pl.BlockSpec((1, D, F), lambda i, s: (s[i], 0, 0)),
            pl.BlockSpec((1, 1, F), lambda i, s: (s[i], 0, 0)),
            pl.BlockSpec((1, F, D), lambda i, s: (s[i], 0, 0)),
            pl.BlockSpec((1, 1, D), lambda i, s: (s[i], 0, 0)),
        ],
        out_specs=pl.BlockSpec((BLK, D), lambda i, s: (i, 0)),
    )
    return pl.pallas_call(
        _gemm_body,
        grid_spec=grid_spec,
        out_shape=jax.ShapeDtypeStruct((NPAD, D), jnp.float32),
    )(s, x16, pos2d, W1, b1.reshape(E, 1, F), W2, b2.reshape(E, 1, D))


@functools.lru_cache(maxsize=None)
def _sc_combine():
    mesh = plsc.VectorSubcoreMesh(core_axis_name="c", subcore_axis_name="s")

    nch = 4
    ch = CCH // nch

    @functools.partial(
        pl.kernel,
        mesh=mesh,
        out_type=jax.ShapeDtypeStruct((T, D), jnp.float32),
        scratch_types=[
            pltpu.VMEM((CCH,), jnp.int32),
            pltpu.VMEM((CCH,), jnp.int32),
            pltpu.VMEM((CCH + 16,), jnp.float32),
            pltpu.VMEM((CCH + 16,), jnp.float32),
            pltpu.VMEM((CCH, D), jnp.float32),
            pltpu.VMEM((CCH, D), jnp.float32),
        ] + [pltpu.SemaphoreType.DMA] * (2 * nch),
    )
    def combine(h_hbm, pos_hbm, gj_hbm, out_hbm,
                i1_v, i2_v, g1_v, g2_v, a_v, b_v, *sems):
        wid = lax.axis_index("s") * 2 + lax.axis_index("c")
        base = wid * CCH
        pltpu.sync_copy(pos_hbm.at[0, pl.ds(base, CCH)], i1_v)
        pltpu.sync_copy(pos_hbm.at[1, pl.ds(base, CCH)], i2_v)
        copies = []
        for c in range(nch):
            sl = pl.ds(c * ch, ch)
            copies.append(pltpu.async_copy(
                h_hbm.at[i1_v.at[sl]], a_v.at[sl], sems[c]))
            copies.append(pltpu.async_copy(
                h_hbm.at[i2_v.at[sl]], b_v.at[sl], sems[nch + c]))
        pltpu.sync_copy(gj_hbm.at[0, pl.ds(base, CCH)], g1_v.at[pl.ds(0, CCH)])
        pltpu.sync_copy(gj_hbm.at[1, pl.ds(base, CCH)], g2_v.at[pl.ds(0, CCH)])
        for cp in copies:
            cp.wait()

        def row_add(i, carry):
            ga = g1_v[pl.ds(i, 16)][0]
            gb = g2_v[pl.ds(i, 16)][0]
            for k in range(D // 16):
                sl = pl.ds(k * 16, 16)
                a_v[i, sl] = a_v[i, sl] * ga + b_v[i, sl] * gb
            return carry

        lax.fori_loop(0, CCH, row_add, 0)
        pltpu.sync_copy(a_v, out_hbm.at[pl.ds(base, CCH)])

    return combine


def kernel(x, Wg, bg, W1, b1, W2, b2):
    b, t, d = x.shape
    xf = x.reshape(T, D)
    pos, gj, s, x16 = _router_call(xf, Wg, bg)
    H = _gemm_call(s.reshape(NB + 1), x16, pos, W1, b1, W2, b2)
    out = _sc_combine()(H, pos, gj)
    return out.reshape(b, t, d)
